# baseline (speedup 1.0000x reference)
import functools

import jax
import jax.numpy as jnp
from jax import lax
from jax.experimental import pallas as pl
from jax.experimental.pallas import tpu as pltpu
from jax.experimental.pallas import tpu_sc as plsc

_B = 128
_DIN = 512
_H1 = 512
_H2 = 1024
_E = 256
_N = 128
_F = 18
_EN = _E * _N
_EHA = 20480
_EHB = 8192
_EHC = _EN - _EHA - _EHB
_FD = _E * (_F - 2)

_CBLK = 2048
_HBLKA = _EHA // _CBLK
_HBLKB = _EHB // _CBLK
_HBLKC = _EHC // _CBLK
_FBLKC = _FD // _HBLKC

_NW = 32
_CHUNK = 128
_GRP = _CHUNK // 16
_U = 4
_SEG = _N // _U


def _tc_first_body(x_ref, ew0_ref, eb0_ref, ew1_ref, eb1_ref, ew2_ref,
                   eb2_ref, fw0_ref, fb0_ref, fw1_ref, fb1_ref,
                   logits_ref, h2e_out_ref, h2f_out_ref, h2e_ref):
    i = pl.program_id(0)

    @pl.when(i == 0)
    def _():
        xv = x_ref[...]
        h1e = jnp.maximum(
            jnp.dot(xv, ew0_ref[...], preferred_element_type=jnp.float32)
            + eb0_ref[...], 0.0)
        h2e = jnp.maximum(
            jnp.dot(h1e, ew1_ref[...], preferred_element_type=jnp.float32)
            + eb1_ref[...], 0.0)
        h2e_ref[...] = h2e
        h2e_out_ref[...] = h2e
        h1f = jnp.maximum(
            jnp.dot(xv, fw0_ref[...], preferred_element_type=jnp.float32)
            + fb0_ref[...], 0.0)
        h2f_out_ref[...] = jnp.maximum(
            jnp.dot(h1f, fw1_ref[...], preferred_element_type=jnp.float32)
            + fb1_ref[...], 0.0)

    logits_ref[...] = (
        jnp.dot(h2e_ref[...], ew2_ref[...], preferred_element_type=jnp.float32)
        + eb2_ref[...])


def _tc_first(x, ew0, eb0, ew1, eb1, ew2, eb2, fw0, fb0, fw1, fb1):
    full = lambda shape: pl.BlockSpec(shape, lambda i: (0, 0))
    return pl.pallas_call(
        _tc_first_body,
        grid=(_HBLKA,),
        in_specs=[
            full((_B, _DIN)),
            full((_DIN, _H1)), full((1, _H1)),
            full((_H1, _H2)), full((1, _H2)),
            pl.BlockSpec((_H2, _CBLK), lambda i: (0, i)),
            pl.BlockSpec((1, _CBLK), lambda i: (0, i)),
            full((_DIN, _H1)), full((1, _H1)),
            full((_H1, _H2)), full((1, _H2)),
        ],
        out_specs=[
            pl.BlockSpec((_B, _CBLK), lambda i: (0, i)),
            full((_B, _H2)),
            full((_B, _H2)),
        ],
        out_shape=[
            jax.ShapeDtypeStruct((_B, _EHA), jnp.float32),
            jax.ShapeDtypeStruct((_B, _H2), jnp.float32),
            jax.ShapeDtypeStruct((_B, _H2), jnp.float32),
        ],
        scratch_shapes=[
            pltpu.VMEM((_B, _H2), jnp.float32),
        ],
        compiler_params=pltpu.CompilerParams(
            dimension_semantics=("arbitrary",)),
    )(x, ew0, eb0, ew1, eb1, ew2, eb2, fw0, fb0, fw1, fb1)


def _tc_second_body(h2e_ref, ew2_ref, eb2_ref, logits_ref):
    logits_ref[...] = (
        jnp.dot(h2e_ref[...], ew2_ref[...], preferred_element_type=jnp.float32)
        + eb2_ref[...])


def _tc_second(h2e, ew2, eb2):
    return pl.pallas_call(
        _tc_second_body,
        grid=(_HBLKB,),
        in_specs=[
            pl.BlockSpec((_B, _H2), lambda i: (0, 0)),
            pl.BlockSpec((_H2, _CBLK), lambda i: (0, i + _HBLKA)),
            pl.BlockSpec((1, _CBLK), lambda i: (0, i + _HBLKA)),
        ],
        out_specs=[pl.BlockSpec((_B, _CBLK), lambda i: (0, i))],
        out_shape=[jax.ShapeDtypeStruct((_B, _EHB), jnp.float32)],
        compiler_params=pltpu.CompilerParams(
            dimension_semantics=("arbitrary",)),
    )(h2e, ew2, eb2)[0]


def _tc_third_body(h2e_ref, ew2_ref, eb2_ref, h2f_ref, fw2_ref, fb2_ref,
                   logits_ref, feat_ref):
    logits_ref[...] = (
        jnp.dot(h2e_ref[...], ew2_ref[...], preferred_element_type=jnp.float32)
        + eb2_ref[...])
    feat_ref[...] = (
        jnp.dot(h2f_ref[...], fw2_ref[...], preferred_element_type=jnp.float32)
        + fb2_ref[...])


def _tc_third(h2e, ew2, eb2, h2f, fw2, fb2):
    return pl.pallas_call(
        _tc_third_body,
        grid=(_HBLKC,),
        in_specs=[
            pl.BlockSpec((_B, _H2), lambda i: (0, 0)),
            pl.BlockSpec((_H2, _CBLK), lambda i: (0, i + _HBLKA + _HBLKB)),
            pl.BlockSpec((1, _CBLK), lambda i: (0, i + _HBLKA + _HBLKB)),
            pl.BlockSpec((_B, _H2), lambda i: (0, 0)),
            pl.BlockSpec((_H2, _FBLKC), lambda i: (0, i)),
            pl.BlockSpec((1, _FBLKC), lambda i: (0, i)),
        ],
        out_specs=[
            pl.BlockSpec((_B, _CBLK), lambda i: (0, i)),
            pl.BlockSpec((_B, _FBLKC), lambda i: (0, i)),
        ],
        out_shape=[
            jax.ShapeDtypeStruct((_B, _EHC), jnp.float32),
            jax.ShapeDtypeStruct((_B, _FD), jnp.float32),
        ],
        compiler_params=pltpu.CompilerParams(
            dimension_semantics=("arbitrary",)),
    )(h2e, ew2, eb2, h2f, fw2, fb2)


def _sc_topk(logits2d, nrows, rpb):
    rpw = nrows // _NW
    nchunk = rpw // _CHUNK
    mesh = plsc.VectorSubcoreMesh(core_axis_name="c", subcore_axis_name="s")

    @functools.partial(
        pl.kernel,
        mesh=mesh,
        out_type=[
            jax.ShapeDtypeStruct((nrows,), jnp.int32),
            jax.ShapeDtypeStruct((nrows,), jnp.int32),
            jax.ShapeDtypeStruct((nrows,), jnp.float32),
            jax.ShapeDtypeStruct((nrows,), jnp.float32),
        ],
        scratch_types=[
            pltpu.VMEM((_CHUNK, _N), jnp.float32),
            pltpu.VMEM((_CHUNK, _N), jnp.float32),
            pltpu.VMEM((rpw,), jnp.int32),
            pltpu.VMEM((rpw,), jnp.int32),
            pltpu.VMEM((rpw,), jnp.float32),
            pltpu.VMEM((rpw,), jnp.float32),
            pltpu.SemaphoreType.DMA,
            pltpu.SemaphoreType.DMA,
        ],
        compiler_params=pltpu.CompilerParams(needs_layout_passes=False),
    )
    def k(lg_hbm, snd_hbm, rcv_hbm, p1_hbm, p2_hbm,
          buf0, buf1, snd_v, rcv_v, p1_v, p2_v, sem0, sem1):
        wid = lax.axis_index("s") * 2 + lax.axis_index("c")
        row0 = wid * rpw
        iota = lax.iota(jnp.int32, 16)
        neg = jnp.full((16,), -3.0e38, jnp.float32)
        zi = jnp.zeros((16,), jnp.int32)
        zf = jnp.zeros((16,), jnp.float32)

        def in_copy(cc, buf, sem):
            return pltpu.make_async_copy(
                lg_hbm.at[pl.ds(row0 + cc * _CHUNK, _CHUNK), :], buf, sem)

        def merge(a, b):
            abv, abi, asv, asi = a
            bbv, bbi, bsv, bsi = b
            gt = bbv > abv
            bv = jnp.where(gt, bbv, abv)
            bi = jnp.where(gt, bbi, abi)
            lv = jnp.where(gt, abv, bbv)
            li = jnp.where(gt, abi, bbi)
            wv = jnp.where(gt, bsv, asv)
            wi = jnp.where(gt, bsi, asi)
            gt2 = wv > lv
            sv = jnp.where(gt2, wv, lv)
            si = jnp.where(gt2, wi, li)
            return bv, bi, sv, si

        def process(cc, buf):
            base = row0 + cc * _CHUNK

            def grp_body(g, carry1):
                rows = g * 16 + iota

                def scan1(kk, carry):
                    rot = (zi + kk + iota) & (_SEG - 1)
                    new = []
                    for j in range(_U):
                        bv, bi, sv, si = carry[4 * j: 4 * j + 4]
                        kv = rot + (_SEG * j)
                        v = plsc.load_gather(buf, [rows, kv])
                        gt1 = v > bv
                        gt2 = v > sv
                        nsv = jnp.where(gt1, bv, jnp.where(gt2, v, sv))
                        nsi = jnp.where(gt1, bi, jnp.where(gt2, kv, si))
                        nbv = jnp.where(gt1, v, bv)
                        nbi = jnp.where(gt1, kv, bi)
                        new += [nbv, nbi, nsv, nsi]
                    return tuple(new)

                st = lax.fori_loop(0, _SEG, scan1, (neg, zi, neg, zi) * _U)
                m01 = merge(st[0:4], st[4:8])
                m23 = merge(st[8:12], st[12:16])
                bv, bi, sv, si = merge(m01, m23)

                def scan2(kk, acc):
                    rot = (zi + kk + iota) & (_SEG - 1)
                    out = []
                    for j in range(_U):
                        v = plsc.load_gather(buf, [rows, rot + _SEG * j])
                        out.append(acc[j] + jnp.exp(v - bv))
                    return tuple(out)

                s0, s1, s2, s3 = lax.fori_loop(0, _SEG, scan2, (zf,) * _U)
                s = (s0 + s1) + (s2 + s3)
                p1 = 1.0 / s
                p2 = jnp.exp(sv - bv) / s
                boff = ((base + g * 16) // rpb) * _N
                off = cc * _CHUNK + g * 16
                snd_v[pl.ds(off, 16)] = bi + boff
                rcv_v[pl.ds(off, 16)] = si + boff
                p1_v[pl.ds(off, 16)] = p1
                p2_v[pl.ds(off, 16)] = p2
                return carry1

            lax.fori_loop(0, _GRP, grp_body, 0)

        in_copy(0, buf0, sem0).start()

        if nchunk >= 2:
            def pair_body(t, carry0):
                c0 = 2 * t
                in_copy(c0, buf0, sem0).wait()
                in_copy(c0 + 1, buf1, sem1).start()
                process(c0, buf0)
                in_copy(c0 + 1, buf1, sem1).wait()

                @pl.when(c0 + 2 < nchunk)
                def _():
                    in_copy(c0 + 2, buf0, sem0).start()

                process(c0 + 1, buf1)
                return carry0

            lax.fori_loop(0, nchunk // 2, pair_body, 0)
        if nchunk % 2:
            in_copy(nchunk - 1, buf0, sem0).wait()
            process(nchunk - 1, buf0)
        pltpu.sync_copy(snd_v, snd_hbm.at[pl.ds(row0, rpw)])
        pltpu.sync_copy(rcv_v, rcv_hbm.at[pl.ds(row0, rpw)])
        pltpu.sync_copy(p1_v, p1_hbm.at[pl.ds(row0, rpw)])
        pltpu.sync_copy(p2_v, p2_hbm.at[pl.ds(row0, rpw)])

    return k(logits2d)


def kernel(x, ew0, eb0, ew1, eb1, ew2, eb2, fw0, fb0, fw1, fb1, fw2, fb2):
    eb2r = eb2.reshape(1, _EN)
    logits_a, h2e, h2f = _tc_first(
        x,
        ew0, eb0.reshape(1, _H1), ew1, eb1.reshape(1, _H2),
        ew2, eb2r,
        fw0, fb0.reshape(1, _H1), fw1, fb1.reshape(1, _H2))
    rpb_a = _EHA // _N
    rpb_b = _EHB // _N
    rpb_c = _EHC // _N
    snd_a, rcv_a, p1_a, p2_a = _sc_topk(
        logits_a.reshape(_B * rpb_a, _N), _B * rpb_a, rpb_a)
    logits_b = _tc_second(h2e, ew2, eb2r)
    snd_b, rcv_b, p1_b, p2_b = _sc_topk(
        logits_b.reshape(_B * rpb_b, _N), _B * rpb_b, rpb_b)
    logits_c, feat = _tc_third(h2e, ew2, eb2r, h2f, fw2, fb2.reshape(1, _FD))
    snd_c, rcv_c, p1_c, p2_c = _sc_topk(
        logits_c.reshape(_B * rpb_c, _N), _B * rpb_c, rpb_c)

    def parts(va, vb, vc):
        return jnp.concatenate(
            [va.reshape(_B, rpb_a), vb.reshape(_B, rpb_b),
             vc.reshape(_B, rpb_c)], axis=1)

    senders = parts(snd_a, snd_b, snd_c)
    receivers = parts(rcv_a, rcv_b, rcv_c)
    probs = jnp.stack(
        [parts(p1_a, p1_b, p1_c), parts(p2_a, p2_b, p2_c)], axis=-1)
    features = jnp.concatenate([probs, feat.reshape(_B, _E, _F - 2)], axis=-1)
    return senders, receivers, features


# speedup vs baseline: 1.0398x; 1.0398x over previous
import functools

import jax
import jax.numpy as jnp
from jax import lax
from jax.experimental import pallas as pl
from jax.experimental.pallas import tpu as pltpu
from jax.experimental.pallas import tpu_sc as plsc

_B = 128
_DIN = 512
_H1 = 512
_H2 = 1024
_E = 256
_N = 128
_F = 18
_EN = _E * _N
_EHA = 24576
_EHB = _EN - _EHA
_FD = _E * (_F - 2)

_CBLK = 2048
_HBLKA = _EHA // _CBLK
_HBLKB = _EHB // _CBLK
_FBLKB = _FD // _HBLKB

_NW = 32
_CHUNK = 128
_GRP = _CHUNK // 16
_U = 4
_SEG = _N // _U
_UNROLL = 4


def _tc_first_body(x_ref, ew0_ref, eb0_ref, ew1_ref, eb1_ref, ew2_ref,
                   eb2_ref, fw0_ref, fb0_ref, fw1_ref, fb1_ref,
                   logits_ref, h2e_out_ref, h2f_out_ref, h2e_ref):
    i = pl.program_id(0)

    @pl.when(i == 0)
    def _():
        xv = x_ref[...]
        h1e = jnp.maximum(
            jnp.dot(xv, ew0_ref[...], preferred_element_type=jnp.float32)
            + eb0_ref[...], 0.0)
        h2e = jnp.maximum(
            jnp.dot(h1e, ew1_ref[...], preferred_element_type=jnp.float32)
            + eb1_ref[...], 0.0)
        h2e_ref[...] = h2e
        h2e_out_ref[...] = h2e
        h1f = jnp.maximum(
            jnp.dot(xv, fw0_ref[...], preferred_element_type=jnp.float32)
            + fb0_ref[...], 0.0)
        h2f_out_ref[...] = jnp.maximum(
            jnp.dot(h1f, fw1_ref[...], preferred_element_type=jnp.float32)
            + fb1_ref[...], 0.0)

    logits_ref[...] = (
        jnp.dot(h2e_ref[...], ew2_ref[...], preferred_element_type=jnp.float32)
        + eb2_ref[...])


def _tc_first(x, ew0, eb0, ew1, eb1, ew2, eb2, fw0, fb0, fw1, fb1):
    full = lambda shape: pl.BlockSpec(shape, lambda i: (0, 0))
    return pl.pallas_call(
        _tc_first_body,
        grid=(_HBLKA,),
        in_specs=[
            full((_B, _DIN)),
            full((_DIN, _H1)), full((1, _H1)),
            full((_H1, _H2)), full((1, _H2)),
            pl.BlockSpec((_H2, _CBLK), lambda i: (0, i)),
            pl.BlockSpec((1, _CBLK), lambda i: (0, i)),
            full((_DIN, _H1)), full((1, _H1)),
            full((_H1, _H2)), full((1, _H2)),
        ],
        out_specs=[
            pl.BlockSpec((_B, _CBLK), lambda i: (0, i)),
            full((_B, _H2)),
            full((_B, _H2)),
        ],
        out_shape=[
            jax.ShapeDtypeStruct((_B, _EHA), jnp.float32),
            jax.ShapeDtypeStruct((_B, _H2), jnp.float32),
            jax.ShapeDtypeStruct((_B, _H2), jnp.float32),
        ],
        scratch_shapes=[
            pltpu.VMEM((_B, _H2), jnp.float32),
        ],
        compiler_params=pltpu.CompilerParams(
            dimension_semantics=("arbitrary",)),
    )(x, ew0, eb0, ew1, eb1, ew2, eb2, fw0, fb0, fw1, fb1)


def _tc_second_body(h2e_ref, ew2_ref, eb2_ref, h2f_ref, fw2_ref, fb2_ref,
                    logits_ref, feat_ref):
    logits_ref[...] = (
        jnp.dot(h2e_ref[...], ew2_ref[...], preferred_element_type=jnp.float32)
        + eb2_ref[...])
    feat_ref[...] = (
        jnp.dot(h2f_ref[...], fw2_ref[...], preferred_element_type=jnp.float32)
        + fb2_ref[...])


def _tc_second(h2e, ew2, eb2, h2f, fw2, fb2):
    return pl.pallas_call(
        _tc_second_body,
        grid=(_HBLKB,),
        in_specs=[
            pl.BlockSpec((_B, _H2), lambda i: (0, 0)),
            pl.BlockSpec((_H2, _CBLK), lambda i: (0, i + _HBLKA)),
            pl.BlockSpec((1, _CBLK), lambda i: (0, i + _HBLKA)),
            pl.BlockSpec((_B, _H2), lambda i: (0, 0)),
            pl.BlockSpec((_H2, _FBLKB), lambda i: (0, i)),
            pl.BlockSpec((1, _FBLKB), lambda i: (0, i)),
        ],
        out_specs=[
            pl.BlockSpec((_B, _CBLK), lambda i: (0, i)),
            pl.BlockSpec((_B, _FBLKB), lambda i: (0, i)),
        ],
        out_shape=[
            jax.ShapeDtypeStruct((_B, _EHB), jnp.float32),
            jax.ShapeDtypeStruct((_B, _FD), jnp.float32),
        ],
        compiler_params=pltpu.CompilerParams(
            dimension_semantics=("arbitrary",)),
    )(h2e, ew2, eb2, h2f, fw2, fb2)


def _sc_topk(logits2d, nrows, rpb):
    rpw = nrows // _NW
    nchunk = rpw // _CHUNK
    mesh = plsc.VectorSubcoreMesh(core_axis_name="c", subcore_axis_name="s")

    @functools.partial(
        pl.kernel,
        mesh=mesh,
        out_type=[
            jax.ShapeDtypeStruct((nrows,), jnp.int32),
            jax.ShapeDtypeStruct((nrows,), jnp.int32),
            jax.ShapeDtypeStruct((nrows,), jnp.float32),
            jax.ShapeDtypeStruct((nrows,), jnp.float32),
        ],
        scratch_types=[
            pltpu.VMEM((_CHUNK, _N), jnp.float32),
            pltpu.VMEM((_CHUNK, _N), jnp.float32),
            pltpu.VMEM((rpw,), jnp.int32),
            pltpu.VMEM((rpw,), jnp.int32),
            pltpu.VMEM((rpw,), jnp.float32),
            pltpu.VMEM((rpw,), jnp.float32),
            pltpu.SemaphoreType.DMA,
            pltpu.SemaphoreType.DMA,
        ],
        compiler_params=pltpu.CompilerParams(needs_layout_passes=False),
    )
    def k(lg_hbm, snd_hbm, rcv_hbm, p1_hbm, p2_hbm,
          buf0, buf1, snd_v, rcv_v, p1_v, p2_v, sem0, sem1):
        wid = lax.axis_index("s") * 2 + lax.axis_index("c")
        row0 = wid * rpw
        iota = lax.iota(jnp.int32, 16)
        neg = jnp.full((16,), -3.0e38, jnp.float32)
        zi = jnp.zeros((16,), jnp.int32)
        zf = jnp.zeros((16,), jnp.float32)

        def in_copy(cc, buf, sem):
            return pltpu.make_async_copy(
                lg_hbm.at[pl.ds(row0 + cc * _CHUNK, _CHUNK), :], buf, sem)

        def merge(a, b):
            abv, abi, asv, asi = a
            bbv, bbi, bsv, bsi = b
            gt = bbv > abv
            bv = jnp.where(gt, bbv, abv)
            bi = jnp.where(gt, bbi, abi)
            lv = jnp.where(gt, abv, bbv)
            li = jnp.where(gt, abi, bbi)
            wv = jnp.where(gt, bsv, asv)
            wi = jnp.where(gt, bsi, asi)
            gt2 = wv > lv
            sv = jnp.where(gt2, wv, lv)
            si = jnp.where(gt2, wi, li)
            return bv, bi, sv, si

        def process(cc, buf):
            base = row0 + cc * _CHUNK

            def grp_body(g, carry1):
                rows = g * 16 + iota

                def scan1(kk, carry):
                    carry = list(carry)
                    for u in range(_UNROLL):
                        rot = (iota + (kk * _UNROLL + u)) & (_SEG - 1)
                        for j in range(_U):
                            bv, bi, sv, si = carry[4 * j: 4 * j + 4]
                            kv = rot + (_SEG * j)
                            v = plsc.load_gather(buf, [rows, kv])
                            gt1 = v > bv
                            gt2 = v > sv
                            nsv = jnp.where(gt1, bv,
                                            jnp.where(gt2, v, sv))
                            nsi = jnp.where(gt1, bi,
                                            jnp.where(gt2, kv, si))
                            carry[4 * j] = jnp.maximum(v, bv)
                            carry[4 * j + 1] = jnp.where(gt1, kv, bi)
                            carry[4 * j + 2] = nsv
                            carry[4 * j + 3] = nsi
                    return tuple(carry)

                st = lax.fori_loop(0, _SEG // _UNROLL, scan1,
                                   (neg, zi, neg, zi) * _U)
                m01 = merge(st[0:4], st[4:8])
                m23 = merge(st[8:12], st[12:16])
                bv, bi, sv, si = merge(m01, m23)

                def scan2(kk, acc):
                    acc = list(acc)
                    for u in range(_UNROLL):
                        rot = (iota + (kk * _UNROLL + u)) & (_SEG - 1)
                        for j in range(_U):
                            v = plsc.load_gather(buf, [rows, rot + _SEG * j])
                            acc[j] = acc[j] + jnp.exp(v - bv)
                    return tuple(acc)

                s0, s1, s2, s3 = lax.fori_loop(0, _SEG // _UNROLL, scan2,
                                               (zf,) * _U)
                s = (s0 + s1) + (s2 + s3)
                p1 = 1.0 / s
                p2 = jnp.exp(sv - bv) / s
                boff = ((base + g * 16) // rpb) * _N
                off = cc * _CHUNK + g * 16
                snd_v[pl.ds(off, 16)] = bi + boff
                rcv_v[pl.ds(off, 16)] = si + boff
                p1_v[pl.ds(off, 16)] = p1
                p2_v[pl.ds(off, 16)] = p2
                return carry1

            lax.fori_loop(0, _GRP, grp_body, 0)

        in_copy(0, buf0, sem0).start()

        if nchunk >= 2:
            def pair_body(t, carry0):
                c0 = 2 * t
                in_copy(c0, buf0, sem0).wait()
                in_copy(c0 + 1, buf1, sem1).start()
                process(c0, buf0)
                in_copy(c0 + 1, buf1, sem1).wait()

                @pl.when(c0 + 2 < nchunk)
                def _():
                    in_copy(c0 + 2, buf0, sem0).start()

                process(c0 + 1, buf1)
                return carry0

            lax.fori_loop(0, nchunk // 2, pair_body, 0)
        if nchunk % 2:
            in_copy(nchunk - 1, buf0, sem0).wait()
            process(nchunk - 1, buf0)
        pltpu.sync_copy(snd_v, snd_hbm.at[pl.ds(row0, rpw)])
        pltpu.sync_copy(rcv_v, rcv_hbm.at[pl.ds(row0, rpw)])
        pltpu.sync_copy(p1_v, p1_hbm.at[pl.ds(row0, rpw)])
        pltpu.sync_copy(p2_v, p2_hbm.at[pl.ds(row0, rpw)])

    return k(logits2d)


def kernel(x, ew0, eb0, ew1, eb1, ew2, eb2, fw0, fb0, fw1, fb1, fw2, fb2):
    eb2r = eb2.reshape(1, _EN)
    logits_a, h2e, h2f = _tc_first(
        x,
        ew0, eb0.reshape(1, _H1), ew1, eb1.reshape(1, _H2),
        ew2, eb2r,
        fw0, fb0.reshape(1, _H1), fw1, fb1.reshape(1, _H2))
    rpb_a = _EHA // _N
    rpb_b = _EHB // _N
    snd_a, rcv_a, p1_a, p2_a = _sc_topk(
        logits_a.reshape(_B * rpb_a, _N), _B * rpb_a, rpb_a)
    logits_b, feat = _tc_second(h2e, ew2, eb2r, h2f, fw2,
                                fb2.reshape(1, _FD))
    snd_b, rcv_b, p1_b, p2_b = _sc_topk(
        logits_b.reshape(_B * rpb_b, _N), _B * rpb_b, rpb_b)

    def halves(va, vb):
        return jnp.concatenate(
            [va.reshape(_B, rpb_a), vb.reshape(_B, rpb_b)], axis=1)

    senders = halves(snd_a, snd_b)
    receivers = halves(rcv_a, rcv_b)
    probs = jnp.stack([halves(p1_a, p1_b), halves(p2_a, p2_b)], axis=-1)
    features = jnp.concatenate([probs, feat.reshape(_B, _E, _F - 2)], axis=-1)
    return senders, receivers, features
